# Initial kernel scaffold; baseline (speedup 1.0000x reference)
import jax
import jax.numpy as jnp
from jax import lax
from jax.experimental import pallas as pl
from jax.experimental.pallas import tpu as pltpu
from jax.experimental.pallas import tpu_sc as plsc

E, K, N = 16384, 8, 50000
D_IMG, D_TXT, H = 256, 512, 512
THRESHOLD = 0.5

NC, NS, L = 2, 16, 16
NW = NC * NS
EPW = E // NW
CHUNK = 16
NCHUNK = EPW // CHUNK


def _pool_body(he_hbm, img_hbm, txt_hbm, out_img_hbm, out_txt_hbm,
               idx_v, rows_img_v, rows_txt_v, pooled_img_v, pooled_txt_v,
               sem_i, sem_t):
    wid = lax.axis_index("s") * NC + lax.axis_index("c")
    base_e = wid * EPW

    def chunk_body(c, carry):
        e0 = base_e + c * CHUNK
        pltpu.sync_copy(he_hbm.at[pl.ds(e0 * K, CHUNK * K)], idx_v)
        cp_i = pltpu.async_copy(img_hbm.at[idx_v], rows_img_v, sem_i)
        cp_t = pltpu.async_copy(txt_hbm.at[idx_v], rows_txt_v, sem_t)
        cp_i.wait()
        cp_t.wait()

        def edge_body(e, carry2):
            r0 = e * K
            for v in range(D_IMG // L):
                sl = pl.ds(v * L, L)
                acc = rows_img_v[r0, sl]
                for k in range(1, K):
                    acc = acc + rows_img_v[r0 + k, sl]
                pooled_img_v[e, sl] = acc * (1.0 / K)
            for v in range(D_TXT // L):
                sl = pl.ds(v * L, L)
                acc = rows_txt_v[r0, sl]
                for k in range(1, K):
                    acc = acc + rows_txt_v[r0 + k, sl]
                pooled_txt_v[e, sl] = acc * (1.0 / K)
            return carry2

        lax.fori_loop(0, CHUNK, edge_body, 0)
        pltpu.sync_copy(pooled_img_v, out_img_hbm.at[pl.ds(e0, CHUNK), :])
        pltpu.sync_copy(pooled_txt_v, out_txt_hbm.at[pl.ds(e0, CHUNK), :])
        return carry

    lax.fori_loop(0, NCHUNK, chunk_body, 0)


_pool = pl.kernel(
    _pool_body,
    out_type=[
        jax.ShapeDtypeStruct((E, D_IMG), jnp.float32),
        jax.ShapeDtypeStruct((E, D_TXT), jnp.float32),
    ],
    mesh=plsc.VectorSubcoreMesh(
        core_axis_name="c", subcore_axis_name="s",
        num_cores=NC, num_subcores=NS),
    scratch_types=[
        pltpu.VMEM((CHUNK * K,), jnp.int32),
        pltpu.VMEM((CHUNK * K, D_IMG), jnp.float32),
        pltpu.VMEM((CHUNK * K, D_TXT), jnp.float32),
        pltpu.VMEM((CHUNK, D_IMG), jnp.float32),
        pltpu.VMEM((CHUNK, D_TXT), jnp.float32),
        pltpu.SemaphoreType.DMA,
        pltpu.SemaphoreType.DMA,
    ],
)

BE = 2048


def _mlp_body(pi_ref, pt_ref, w1i_ref, b1i_ref, w2i_ref,
              w1t_ref, b1t_ref, w2t_ref, scal_ref,
              scores_ref, mask_ref):
    hi = jnp.maximum(
        jnp.dot(pi_ref[...], w1i_ref[...],
                preferred_element_type=jnp.float32,
                precision=lax.Precision.HIGHEST) + b1i_ref[...], 0.0)
    si = jnp.sum(hi * w2i_ref[...], axis=1) + scal_ref[0]
    ht = jnp.maximum(
        jnp.dot(pt_ref[...], w1t_ref[...],
                preferred_element_type=jnp.float32,
                precision=lax.Precision.HIGHEST) + b1t_ref[...], 0.0)
    st = jnp.sum(ht * w2t_ref[...], axis=1) + scal_ref[1]
    e_score = scal_ref[2] * si + scal_ref[3] * st
    scores = jax.nn.sigmoid(e_score)
    scores_ref[...] = scores
    mask_ref[...] = scores > THRESHOLD


_mlp = pl.pallas_call(
    _mlp_body,
    grid=(E // BE,),
    in_specs=[
        pl.BlockSpec((BE, D_IMG), lambda i: (i, 0)),
        pl.BlockSpec((BE, D_TXT), lambda i: (i, 0)),
        pl.BlockSpec((D_IMG, H), lambda i: (0, 0)),
        pl.BlockSpec((1, H), lambda i: (0, 0)),
        pl.BlockSpec((1, H), lambda i: (0, 0)),
        pl.BlockSpec((D_TXT, H), lambda i: (0, 0)),
        pl.BlockSpec((1, H), lambda i: (0, 0)),
        pl.BlockSpec((1, H), lambda i: (0, 0)),
        pl.BlockSpec(memory_space=pltpu.SMEM),
    ],
    out_specs=[
        pl.BlockSpec((BE,), lambda i: (i,)),
        pl.BlockSpec((BE,), lambda i: (i,)),
    ],
    out_shape=[
        jax.ShapeDtypeStruct((E,), jnp.float32),
        jax.ShapeDtypeStruct((E,), jnp.bool_),
    ],
)


def kernel(hyperedges, features_image, features_text,
           W1_image, b1_image, W2_image, b2_image,
           W1_text, b1_text, W2_text, b2_text, alpha):
    he = jnp.asarray(hyperedges, jnp.int32).reshape(E * K)
    pooled_img, pooled_txt = _pool(he, features_image, features_text)

    w = jax.nn.softmax(alpha, axis=0)
    scal = jnp.stack([b2_image[0], b2_text[0], w[0], w[1]])
    scores, mask = _mlp(
        pooled_img, pooled_txt,
        W1_image, b1_image.reshape(1, H), W2_image.reshape(1, H),
        W1_text, b1_text.reshape(1, H), W2_text.reshape(1, H),
        scal)
    return (mask, scores)


# speedup vs baseline: 1.4873x; 1.4873x over previous
import functools

import jax
import jax.numpy as jnp
from jax import lax
from jax.experimental import pallas as pl
from jax.experimental.pallas import tpu as pltpu
from jax.experimental.pallas import tpu_sc as plsc

E, K, N = 16384, 8, 50000
D_IMG, D_TXT, H = 256, 512, 512
THRESHOLD = 0.5

NC, NS, L = 2, 16, 16
NW = NC * NS
EPW = E // NW
CHUNK = 16
NCHUNK = EPW // CHUNK


def _pool_body(he_hbm, img_hbm, txt_hbm, out_img_hbm, out_txt_hbm,
               idx_v, rows_img_v, rows_txt_v, pooled_img_v, pooled_txt_v,
               sem_i, sem_t):
    wid = lax.axis_index("s") * NC + lax.axis_index("c")
    base_e = wid * EPW

    def chunk_body(c, carry):
        e0 = base_e + c * CHUNK
        pltpu.sync_copy(he_hbm.at[pl.ds(e0 * K, CHUNK * K)], idx_v)
        cp_i = pltpu.async_copy(img_hbm.at[idx_v], rows_img_v, sem_i)
        cp_t = pltpu.async_copy(txt_hbm.at[idx_v], rows_txt_v, sem_t)
        cp_i.wait()
        cp_t.wait()

        def edge_body(e, carry2):
            r0 = e * K
            for v in range(D_IMG // L):
                sl = pl.ds(v * L, L)
                acc = rows_img_v[r0, sl]
                for k in range(1, K):
                    acc = acc + rows_img_v[r0 + k, sl]
                pooled_img_v[e, sl] = acc * (1.0 / K)
            for v in range(D_TXT // L):
                sl = pl.ds(v * L, L)
                acc = rows_txt_v[r0, sl]
                for k in range(1, K):
                    acc = acc + rows_txt_v[r0 + k, sl]
                pooled_txt_v[e, sl] = acc * (1.0 / K)
            return carry2

        lax.fori_loop(0, CHUNK, edge_body, 0)
        pltpu.sync_copy(pooled_img_v, out_img_hbm.at[pl.ds(e0, CHUNK), :])
        pltpu.sync_copy(pooled_txt_v, out_txt_hbm.at[pl.ds(e0, CHUNK), :])
        return carry

    lax.fori_loop(0, NCHUNK, chunk_body, 0)


@functools.cache
def _get_pool():
    return pl.kernel(
        _pool_body,
        out_type=[
            jax.ShapeDtypeStruct((E, D_IMG), jnp.float32),
            jax.ShapeDtypeStruct((E, D_TXT), jnp.float32),
        ],
        mesh=plsc.VectorSubcoreMesh(
            core_axis_name="c", subcore_axis_name="s",
            num_cores=NC, num_subcores=NS),
        scratch_types=[
            pltpu.VMEM((CHUNK * K,), jnp.int32),
            pltpu.VMEM((CHUNK * K, D_IMG), jnp.float32),
            pltpu.VMEM((CHUNK * K, D_TXT), jnp.float32),
            pltpu.VMEM((CHUNK, D_IMG), jnp.float32),
            pltpu.VMEM((CHUNK, D_TXT), jnp.float32),
            pltpu.SemaphoreType.DMA,
            pltpu.SemaphoreType.DMA,
        ],
    )

BE = 2048


def _mlp_body(pi_ref, pt_ref, w1i_ref, b1i_ref, w2i_ref,
              w1t_ref, b1t_ref, w2t_ref, scal_ref,
              scores_ref, mask_ref):
    hi = jnp.maximum(
        jnp.dot(pi_ref[...], w1i_ref[...],
                preferred_element_type=jnp.float32) + b1i_ref[...], 0.0)
    si = jnp.dot(hi, w2i_ref[...],
                 preferred_element_type=jnp.float32)[:, 0] + scal_ref[0]
    ht = jnp.maximum(
        jnp.dot(pt_ref[...], w1t_ref[...],
                preferred_element_type=jnp.float32) + b1t_ref[...], 0.0)
    st = jnp.dot(ht, w2t_ref[...],
                 preferred_element_type=jnp.float32)[:, 0] + scal_ref[1]
    e_score = scal_ref[2] * si + scal_ref[3] * st
    scores = jax.nn.sigmoid(e_score)
    scores_ref[...] = scores
    mask_ref[...] = scores > THRESHOLD


_mlp_in_specs = [
    pl.BlockSpec((BE, D_IMG), lambda i: (i, 0)),
    pl.BlockSpec((BE, D_TXT), lambda i: (i, 0)),
    pl.BlockSpec((D_IMG, H), lambda i: (0, 0)),
    pl.BlockSpec((1, H), lambda i: (0, 0)),
    pl.BlockSpec((H, 1), lambda i: (0, 0)),
    pl.BlockSpec((D_TXT, H), lambda i: (0, 0)),
    pl.BlockSpec((1, H), lambda i: (0, 0)),
    pl.BlockSpec((H, 1), lambda i: (0, 0)),
    pl.BlockSpec(memory_space=pltpu.SMEM),
]
_mlp_out_specs = [
    pl.BlockSpec((BE,), lambda i: (i,)),
    pl.BlockSpec((BE,), lambda i: (i,)),
]
_mlp_out_shape = [
    jax.ShapeDtypeStruct((E,), jnp.float32),
    jax.ShapeDtypeStruct((E,), jnp.bool_),
]

_mlp = pl.pallas_call(
    _mlp_body,
    grid=(E // BE,),
    in_specs=_mlp_in_specs,
    out_specs=_mlp_out_specs,
    out_shape=_mlp_out_shape,
)


def kernel(hyperedges, features_image, features_text,
           W1_image, b1_image, W2_image, b2_image,
           W1_text, b1_text, W2_text, b2_text, alpha):
    he = jnp.asarray(hyperedges, jnp.int32).reshape(E * K)
    pooled_img, pooled_txt = _get_pool()(he, features_image, features_text)

    w = jax.nn.softmax(alpha, axis=0)
    scal = jnp.stack([b2_image[0], b2_text[0], w[0], w[1]])
    scores, mask = _mlp(
        pooled_img, pooled_txt,
        W1_image, b1_image.reshape(1, H), W2_image,
        W1_text, b1_text.reshape(1, H), W2_text,
        scal)
    return (mask, scores)


# speedup vs baseline: 2.1293x; 1.4316x over previous
import functools

import jax
import jax.numpy as jnp
from jax import lax
from jax.experimental import pallas as pl
from jax.experimental.pallas import tpu as pltpu
from jax.experimental.pallas import tpu_sc as plsc

E, K, N = 16384, 8, 50000
D_IMG, D_TXT, H = 256, 512, 512
THRESHOLD = 0.5

NC, NS, L = 2, 16, 16
NW = NC * NS
EPW = E // NW
CHUNK = 8
CK = CHUNK * K
NCHUNK = EPW // CHUNK
HALF = NCHUNK // 2


def _pool_body(he_hbm, img_hbm, txt_hbm, out_img_hbm, out_txt_hbm,
               idx_all, rows_img0, rows_img1, rows_txt0, rows_txt1,
               pooled_img0, pooled_img1, pooled_txt0, pooled_txt1,
               sem_i0, sem_i1, sem_t0, sem_t1,
               sem_oi0, sem_oi1, sem_ot0, sem_ot1):
    wid = lax.axis_index("s") * NC + lax.axis_index("c")
    base_e = wid * EPW
    pltpu.sync_copy(he_hbm.at[pl.ds(base_e * K, EPW * K)], idx_all)

    rows_img = (rows_img0, rows_img1)
    rows_txt = (rows_txt0, rows_txt1)
    pooled_img = (pooled_img0, pooled_img1)
    pooled_txt = (pooled_txt0, pooled_txt1)
    sem_i = (sem_i0, sem_i1)
    sem_t = (sem_t0, sem_t1)
    sem_oi = (sem_oi0, sem_oi1)
    sem_ot = (sem_ot0, sem_ot1)

    def start(c, b):
        idx = idx_all.at[pl.ds(c * CK, CK)]
        pltpu.async_copy(img_hbm.at[idx], rows_img[b], sem_i[b])
        pltpu.async_copy(txt_hbm.at[idx], rows_txt[b], sem_t[b])

    def finish(c, b):
        e0 = base_e + c * CHUNK
        idx = idx_all.at[pl.ds(c * CK, CK)]
        pltpu.make_async_copy(img_hbm.at[idx], rows_img[b], sem_i[b]).wait()
        pltpu.make_async_copy(txt_hbm.at[idx], rows_txt[b], sem_t[b]).wait()

        @pl.when(c >= 2)
        def _():
            pltpu.make_async_copy(
                pooled_img[b], out_img_hbm.at[pl.ds(e0, CHUNK), :],
                sem_oi[b]).wait()
            pltpu.make_async_copy(
                pooled_txt[b], out_txt_hbm.at[pl.ds(e0, CHUNK), :],
                sem_ot[b]).wait()

        def edge_body(e, carry):
            r0 = e * K
            for v in range(D_IMG // L):
                sl = pl.ds(v * L, L)
                acc = rows_img[b][r0, sl]
                for k in range(1, K):
                    acc = acc + rows_img[b][r0 + k, sl]
                pooled_img[b][e, sl] = acc * (1.0 / K)
            for v in range(D_TXT // L):
                sl = pl.ds(v * L, L)
                acc = rows_txt[b][r0, sl]
                for k in range(1, K):
                    acc = acc + rows_txt[b][r0 + k, sl]
                pooled_txt[b][e, sl] = acc * (1.0 / K)
            return carry

        lax.fori_loop(0, CHUNK, edge_body, 0)
        pltpu.async_copy(pooled_img[b], out_img_hbm.at[pl.ds(e0, CHUNK), :],
                         sem_oi[b])
        pltpu.async_copy(pooled_txt[b], out_txt_hbm.at[pl.ds(e0, CHUNK), :],
                         sem_ot[b])

    start(0, 0)
    start(1, 1)

    def body(i, carry):
        c0 = 2 * i
        finish(c0, 0)

        @pl.when(i < HALF - 1)
        def _():
            start(c0 + 2, 0)

        finish(c0 + 1, 1)

        @pl.when(i < HALF - 1)
        def _():
            start(c0 + 3, 1)

        return carry

    lax.fori_loop(0, HALF, body, 0)

    for b, c in ((0, NCHUNK - 2), (1, NCHUNK - 1)):
        e0 = base_e + c * CHUNK
        pltpu.make_async_copy(
            pooled_img[b], out_img_hbm.at[pl.ds(e0, CHUNK), :],
            sem_oi[b]).wait()
        pltpu.make_async_copy(
            pooled_txt[b], out_txt_hbm.at[pl.ds(e0, CHUNK), :],
            sem_ot[b]).wait()


@functools.cache
def _get_pool():
    return pl.kernel(
        _pool_body,
        out_type=[
            jax.ShapeDtypeStruct((E, D_IMG), jnp.float32),
            jax.ShapeDtypeStruct((E, D_TXT), jnp.float32),
        ],
        mesh=plsc.VectorSubcoreMesh(
            core_axis_name="c", subcore_axis_name="s",
            num_cores=NC, num_subcores=NS),
        scratch_types=[
            pltpu.VMEM((EPW * K,), jnp.int32),
            pltpu.VMEM((CK, D_IMG), jnp.float32),
            pltpu.VMEM((CK, D_IMG), jnp.float32),
            pltpu.VMEM((CK, D_TXT), jnp.float32),
            pltpu.VMEM((CK, D_TXT), jnp.float32),
            pltpu.VMEM((CHUNK, D_IMG), jnp.float32),
            pltpu.VMEM((CHUNK, D_IMG), jnp.float32),
            pltpu.VMEM((CHUNK, D_TXT), jnp.float32),
            pltpu.VMEM((CHUNK, D_TXT), jnp.float32),
            pltpu.SemaphoreType.DMA,
            pltpu.SemaphoreType.DMA,
            pltpu.SemaphoreType.DMA,
            pltpu.SemaphoreType.DMA,
            pltpu.SemaphoreType.DMA,
            pltpu.SemaphoreType.DMA,
            pltpu.SemaphoreType.DMA,
            pltpu.SemaphoreType.DMA,
        ],
    )

BE = 2048


def _mlp_body(pi_ref, pt_ref, w1i_ref, b1i_ref, w2i_ref,
              w1t_ref, b1t_ref, w2t_ref, scal_ref,
              scores_ref, mask_ref):
    hi = jnp.maximum(
        jnp.dot(pi_ref[...], w1i_ref[...],
                preferred_element_type=jnp.float32) + b1i_ref[...], 0.0)
    si = jnp.dot(hi, w2i_ref[...],
                 preferred_element_type=jnp.float32)[:, 0] + scal_ref[0]
    ht = jnp.maximum(
        jnp.dot(pt_ref[...], w1t_ref[...],
                preferred_element_type=jnp.float32) + b1t_ref[...], 0.0)
    st = jnp.dot(ht, w2t_ref[...],
                 preferred_element_type=jnp.float32)[:, 0] + scal_ref[1]
    e_score = scal_ref[2] * si + scal_ref[3] * st
    scores = jax.nn.sigmoid(e_score)
    scores_ref[...] = scores
    mask_ref[...] = scores > THRESHOLD


_mlp_in_specs = [
    pl.BlockSpec((BE, D_IMG), lambda i: (i, 0)),
    pl.BlockSpec((BE, D_TXT), lambda i: (i, 0)),
    pl.BlockSpec((D_IMG, H), lambda i: (0, 0)),
    pl.BlockSpec((1, H), lambda i: (0, 0)),
    pl.BlockSpec((H, 1), lambda i: (0, 0)),
    pl.BlockSpec((D_TXT, H), lambda i: (0, 0)),
    pl.BlockSpec((1, H), lambda i: (0, 0)),
    pl.BlockSpec((H, 1), lambda i: (0, 0)),
    pl.BlockSpec(memory_space=pltpu.SMEM),
]
_mlp_out_specs = [
    pl.BlockSpec((BE,), lambda i: (i,)),
    pl.BlockSpec((BE,), lambda i: (i,)),
]
_mlp_out_shape = [
    jax.ShapeDtypeStruct((E,), jnp.float32),
    jax.ShapeDtypeStruct((E,), jnp.bool_),
]

_mlp = pl.pallas_call(
    _mlp_body,
    grid=(E // BE,),
    in_specs=_mlp_in_specs,
    out_specs=_mlp_out_specs,
    out_shape=_mlp_out_shape,
)


def kernel(hyperedges, features_image, features_text,
           W1_image, b1_image, W2_image, b2_image,
           W1_text, b1_text, W2_text, b2_text, alpha):
    he = jnp.asarray(hyperedges, jnp.int32).reshape(E * K)
    pooled_img, pooled_txt = _get_pool()(he, features_image, features_text)

    w = jax.nn.softmax(alpha, axis=0)
    scal = jnp.stack([b2_image[0], b2_text[0], w[0], w[1]])
    scores, mask = _mlp(
        pooled_img, pooled_txt,
        W1_image, b1_image.reshape(1, H), W2_image,
        W1_text, b1_text.reshape(1, H), W2_text,
        scal)
    return (mask, scores)
